# baseline (speedup 1.0000x reference)
import dataclasses
import functools

import jax
import jax.numpy as jnp
from jax import lax
from jax.experimental import pallas as pl
from jax.experimental.pallas import tpu as pltpu
from jax.experimental.pallas import tpu_sc as plsc

N = 10000
E = 320000
D = 128

NC = 2
NS = 16
L = 16
NW = NC * NS
EPW = E // NW
CHA = 128
NCHA = EPW // CHA
CHB = 64
NCHB = EPW // CHB
TAIL = EPW - NCHA * CHA
TAIL_OFF = EPW - TAIL
RPT = N // NS

_mesh = plsc.VectorSubcoreMesh(core_axis_name="c", subcore_axis_name="s")

_sc_params = pltpu.CompilerParams()
if "needs_layout_passes" in pltpu.CompilerParams.__dataclass_fields__:
    _sc_params = dataclasses.replace(_sc_params, needs_layout_passes=False)


def _sc_deg(src_hbm, w_hbm, out_hbm, deg_v, src_v0, src_v1, w_v0, w_v1,
            st_v, wt_v, isem0, isem1):
    c = lax.axis_index("c")
    s = lax.axis_index("s")
    base0 = (c * NS + s) * EPW

    @pl.loop(0, N, step=L)
    def _(r):
        deg_v[pl.ds(r, L)] = jnp.zeros((L,), jnp.float32)

    srcs = (src_v0, src_v1)
    ws = (w_v0, w_v1)
    isems = (isem0, isem1)

    pltpu.sync_copy(src_hbm.at[pl.ds(base0, CHA)], src_v0)
    pltpu.sync_copy(w_hbm.at[pl.ds(base0, CHA)], w_v0)
    pltpu.async_copy(src_hbm.at[pl.ds(base0 + CHA, CHA)], src_v1, isem1)
    pltpu.async_copy(w_hbm.at[pl.ds(base0 + CHA, CHA)], w_v1, isem1)

    @pl.loop(0, NCHA // 2)
    def _(gg):
        for b in (0, 1):
            cix = 2 * gg + b
            nb = 1 - b

            @pl.when(cix + 1 < NCHA)
            def _():
                pltpu.make_async_copy(
                    src_hbm.at[pl.ds(base0, CHA)], srcs[nb], isems[nb]).wait()
                pltpu.make_async_copy(
                    w_hbm.at[pl.ds(base0, CHA)], ws[nb], isems[nb]).wait()

            @pl.loop(0, CHA, step=L)
            def _(j):
                s16 = srcs[b][pl.ds(j, L)]
                w16 = ws[b][pl.ds(j, L)]
                plsc.addupdate_scatter(deg_v, [s16], w16)

            @pl.when(cix + 2 < NCHA)
            def _():
                nbase = base0 + (cix + 2) * CHA
                pltpu.async_copy(src_hbm.at[pl.ds(nbase, CHA)], srcs[b],
                                 isems[b])
                pltpu.async_copy(w_hbm.at[pl.ds(nbase, CHA)], ws[b], isems[b])

    pltpu.sync_copy(src_hbm.at[pl.ds(base0 + TAIL_OFF, TAIL)], st_v)
    pltpu.sync_copy(w_hbm.at[pl.ds(base0 + TAIL_OFF, TAIL)], wt_v)
    plsc.addupdate_scatter(deg_v, [st_v[...]], wt_v[...])

    pltpu.sync_copy(deg_v, out_hbm.at[c, s])


def _sc_edge(src_hbm, dst_hbm, w_hbm, dis_hbm, h_hbm, z_hbm, out_hbm, acc_sh,
             src_v0, src_v1, src_v2, dst_v0, dst_v1, dst_v2,
             w_v0, w_v1, w_v2, wo_v0, wo_v1, wo_v2,
             dsc_v0, dsc_v1, dsc_v2, rows_v0, rows_v1, rows_v2,
             dis_v, st_v, dt_v, wt_v, rt_v,
             isem0, isem1, isem2, gsem0, gsem1, gsem2,
             osem0, osem1, osem2):
    c = lax.axis_index("c")
    s = lax.axis_index("s")
    base0 = (c * NS + s) * EPW

    pltpu.sync_copy(z_hbm, acc_sh.at[pl.ds(s * RPT, RPT)])
    pltpu.sync_copy(dis_hbm.at[0], dis_v)
    plsc.subcore_barrier()

    srcs = (src_v0, src_v1, src_v2)
    dsts = (dst_v0, dst_v1, dst_v2)
    ws = (w_v0, w_v1, w_v2)
    wos = (wo_v0, wo_v1, wo_v2)
    dscs = (dsc_v0, dsc_v1, dsc_v2)
    rows = (rows_v0, rows_v1, rows_v2)
    isems = (isem0, isem1, isem2)
    gsems = (gsem0, gsem1, gsem2)
    osems = (osem0, osem1, osem2)

    def issue_idx(cc, b):
        nbase = base0 + cc * CHB
        pltpu.async_copy(src_hbm.at[pl.ds(nbase, CHB)], srcs[b], isems[b])
        pltpu.async_copy(dst_hbm.at[pl.ds(nbase, CHB)], dsts[b], isems[b])
        pltpu.async_copy(w_hbm.at[pl.ds(nbase, CHB)], ws[b], isems[b])

    def wait_idx(b):
        pltpu.make_async_copy(
            src_hbm.at[pl.ds(base0, CHB)], srcs[b], isems[b]).wait()
        pltpu.make_async_copy(
            dst_hbm.at[pl.ds(base0, CHB)], dsts[b], isems[b]).wait()
        pltpu.make_async_copy(
            w_hbm.at[pl.ds(base0, CHB)], ws[b], isems[b]).wait()

    pltpu.sync_copy(src_hbm.at[pl.ds(base0, CHB)], src_v0)
    pltpu.sync_copy(dst_hbm.at[pl.ds(base0, CHB)], dst_v0)
    pltpu.sync_copy(w_hbm.at[pl.ds(base0, CHB)], w_v0)
    issue_idx(1, 1)
    issue_idx(2, 2)
    pltpu.async_copy(h_hbm.at[src_v0], rows_v0, gsem0)
    wait_idx(1)
    pltpu.async_copy(h_hbm.at[src_v1], rows_v1, gsem1)

    @pl.loop(0, NCHB // 3)
    def _(gg):
        for b in (0, 1, 2):
            cix = 3 * gg + b
            b2 = (b + 2) % 3

            pltpu.make_async_copy(
                h_hbm.at[srcs[b]], rows[b], gsems[b]).wait()

            @plsc.parallel_loop(0, CHB, step=L, unroll=2)
            def _(j):
                s16 = srcs[b][pl.ds(j, L)]
                d16 = dsts[b][pl.ds(j, L)]
                w16 = ws[b][pl.ds(j, L)]
                dis_s = plsc.load_gather(dis_v, [s16])
                dis_d = plsc.load_gather(dis_v, [d16])
                wos[b][pl.ds(j, L)] = -(dis_s * w16 * dis_d)
                dscs[b][pl.ds(j, L)] = d16

            @pl.when(cix >= 1)
            def _():
                pltpu.make_async_copy(
                    rows[b2], acc_sh.at[dscs[b2]], osems[b2]).wait()

            @pl.when(cix + 2 < NCHB)
            def _():
                wait_idx(b2)
                pltpu.async_copy(h_hbm.at[srcs[b2]], rows[b2], gsems[b2])

            @plsc.parallel_loop(0, CHB, step=L, unroll=2)
            def _(j):
                wo16 = wos[b][pl.ds(j, L)]
                for i in range(L):
                    sc = wo16[i]
                    for k in range(D // L):
                        rows[b][j + i, pl.ds(k * L, L)] = (
                            rows[b][j + i, pl.ds(k * L, L)] * sc)

            pltpu.async_copy(rows[b], acc_sh.at[dscs[b]], osems[b],
                             add=True)

            @pl.when(cix + 3 < NCHB)
            def _():
                issue_idx(cix + 3, b)

    pltpu.make_async_copy(rows_v2, acc_sh.at[dsc_v2], osem2).wait()

    pltpu.sync_copy(src_hbm.at[pl.ds(base0 + TAIL_OFF, TAIL)], st_v)
    pltpu.sync_copy(dst_hbm.at[pl.ds(base0 + TAIL_OFF, TAIL)], dt_v)
    pltpu.sync_copy(w_hbm.at[pl.ds(base0 + TAIL_OFF, TAIL)], wt_v)
    pltpu.sync_copy(h_hbm.at[st_v], rt_v)
    s16 = st_v[...]
    d16 = dt_v[...]
    w16 = wt_v[...]
    wo16 = -(plsc.load_gather(dis_v, [s16]) * w16
             * plsc.load_gather(dis_v, [d16]))
    for i in range(L):
        sc = wo16[i]
        for k in range(D // L):
            rt_v[i, pl.ds(k * L, L)] = rt_v[i, pl.ds(k * L, L)] * sc
    pltpu.sync_copy(rt_v, acc_sh.at[dt_v], add=True)

    plsc.subcore_barrier()
    pltpu.sync_copy(acc_sh.at[pl.ds(s * RPT, RPT)], out_hbm.at[c, s])


def _tc_dis(degw_ref, dis_ref):
    deg = jnp.sum(degw_ref[...], axis=0)
    dis_ref[...] = jnp.where(deg > 0, lax.rsqrt(deg), 0.0)[None, :]


BLK = 1000


def _tc_dense(x_ref, h_ref, c_ref, p_ref, wx_ref, w0_ref, w1_ref, b_ref,
              wp_ref, wl_ref, bl_ref, ho_ref, H_ref, C_ref):
    x = x_ref[...]
    hh = h_ref[...]
    cc = c_ref[...]
    tx1 = p_ref[0] + p_ref[1]

    dot = functools.partial(jnp.dot, preferred_element_type=jnp.float32,
                            precision=lax.Precision.HIGHEST)
    pre = (dot(x, wx_ref[...]) + dot(hh, w0_ref[...]) + dot(tx1, w1_ref[...])
           + b_ref[...])
    wp = wp_ref[...]
    gi = jax.nn.sigmoid(pre[:, 0:D] + wp[0:1] * cc)
    gf = jax.nn.sigmoid(pre[:, D:2 * D] + wp[1:2] * cc)
    gt = jnp.tanh(pre[:, 2 * D:3 * D])
    cn = gf * cc + gi * gt
    go = jax.nn.sigmoid(pre[:, 3 * D:4 * D] + wp[2:3] * cn)
    hn = go * jnp.tanh(cn)
    C_ref[...] = cn
    H_ref[...] = hn
    ho_ref[...] = (jnp.sum(jax.nn.relu(hn) * wl_ref[...], axis=1,
                           keepdims=True) + bl_ref[0, 0])


def kernel(x, edge_index, edge_weight, h, c, W_x, conv_W0, conv_W1, conv_b,
           w_peep, b_gate, W_lin, b_lin):
    src = edge_index[0]
    dst = edge_index[1]
    wm = jnp.where(src == dst, 0.0, edge_weight)

    deg_fn = pl.kernel(
        _sc_deg,
        out_type=jax.ShapeDtypeStruct((NC, NS, N), jnp.float32),
        mesh=_mesh,
        scratch_types=[
            pltpu.VMEM((N,), jnp.float32),
            pltpu.VMEM((CHA,), jnp.int32),
            pltpu.VMEM((CHA,), jnp.int32),
            pltpu.VMEM((CHA,), jnp.float32),
            pltpu.VMEM((CHA,), jnp.float32),
            pltpu.VMEM((TAIL,), jnp.int32),
            pltpu.VMEM((TAIL,), jnp.float32),
            pltpu.SemaphoreType.DMA,
            pltpu.SemaphoreType.DMA,
        ],
        compiler_params=_sc_params,
    )
    degw = deg_fn(src, wm).reshape(NC * NS, N)

    dis = pl.pallas_call(
        _tc_dis,
        out_shape=jax.ShapeDtypeStruct((1, N), jnp.float32),
    )(degw)

    zeros_blk = jnp.zeros((RPT, D), jnp.float32)

    edge_fn = pl.kernel(
        _sc_edge,
        out_type=jax.ShapeDtypeStruct((NC, NS, RPT, D), jnp.float32),
        mesh=_mesh,
        scratch_types=[
            pltpu.VMEM_SHARED((N, D), jnp.float32),
            pltpu.VMEM((CHB,), jnp.int32),
            pltpu.VMEM((CHB,), jnp.int32),
            pltpu.VMEM((CHB,), jnp.int32),
            pltpu.VMEM((CHB,), jnp.int32),
            pltpu.VMEM((CHB,), jnp.int32),
            pltpu.VMEM((CHB,), jnp.int32),
            pltpu.VMEM((CHB,), jnp.float32),
            pltpu.VMEM((CHB,), jnp.float32),
            pltpu.VMEM((CHB,), jnp.float32),
            pltpu.VMEM((CHB,), jnp.float32),
            pltpu.VMEM((CHB,), jnp.float32),
            pltpu.VMEM((CHB,), jnp.float32),
            pltpu.VMEM((CHB,), jnp.int32),
            pltpu.VMEM((CHB,), jnp.int32),
            pltpu.VMEM((CHB,), jnp.int32),
            pltpu.VMEM((CHB, D), jnp.float32),
            pltpu.VMEM((CHB, D), jnp.float32),
            pltpu.VMEM((CHB, D), jnp.float32),
            pltpu.VMEM((N,), jnp.float32),
            pltpu.VMEM((TAIL,), jnp.int32),
            pltpu.VMEM((TAIL,), jnp.int32),
            pltpu.VMEM((TAIL,), jnp.float32),
            pltpu.VMEM((TAIL, D), jnp.float32),
            pltpu.SemaphoreType.DMA,
            pltpu.SemaphoreType.DMA,
            pltpu.SemaphoreType.DMA,
            pltpu.SemaphoreType.DMA,
            pltpu.SemaphoreType.DMA,
            pltpu.SemaphoreType.DMA,
            pltpu.SemaphoreType.DMA,
            pltpu.SemaphoreType.DMA,
            pltpu.SemaphoreType.DMA,
        ],
        compiler_params=_sc_params,
    )
    parts = edge_fn(src, dst, wm, dis, h, zeros_blk).reshape(NC, N, D)

    wx_cat = jnp.transpose(W_x, (1, 0, 2)).reshape(D, 4 * D)
    w0_cat = jnp.transpose(conv_W0, (1, 0, 2)).reshape(D, 4 * D)
    w1_cat = jnp.transpose(conv_W1, (1, 0, 2)).reshape(D, 4 * D)
    b_cat = (conv_b + b_gate).reshape(1, 4 * D)
    wl_row = W_lin.reshape(1, D)
    bl = b_lin.reshape(1, 1)

    grid = (N // BLK,)
    h_out, H, C = pl.pallas_call(
        _tc_dense,
        grid=grid,
        in_specs=[
            pl.BlockSpec((BLK, D), lambda i: (i, 0)),
            pl.BlockSpec((BLK, D), lambda i: (i, 0)),
            pl.BlockSpec((BLK, D), lambda i: (i, 0)),
            pl.BlockSpec((NC, BLK, D), lambda i: (0, i, 0)),
            pl.BlockSpec((D, 4 * D), lambda i: (0, 0)),
            pl.BlockSpec((D, 4 * D), lambda i: (0, 0)),
            pl.BlockSpec((D, 4 * D), lambda i: (0, 0)),
            pl.BlockSpec((1, 4 * D), lambda i: (0, 0)),
            pl.BlockSpec((3, D), lambda i: (0, 0)),
            pl.BlockSpec((1, D), lambda i: (0, 0)),
            pl.BlockSpec((1, 1), lambda i: (0, 0)),
        ],
        out_specs=[
            pl.BlockSpec((BLK, 1), lambda i: (i, 0)),
            pl.BlockSpec((BLK, D), lambda i: (i, 0)),
            pl.BlockSpec((BLK, D), lambda i: (i, 0)),
        ],
        out_shape=[
            jax.ShapeDtypeStruct((N, 1), jnp.float32),
            jax.ShapeDtypeStruct((N, D), jnp.float32),
            jax.ShapeDtypeStruct((N, D), jnp.float32),
        ],
    )(x, h, c, parts, wx_cat, w0_cat, w1_cat, b_cat, w_peep, wl_row, bl)

    return (h_out, H, C)


# speedup vs baseline: 1.3756x; 1.3756x over previous
import dataclasses
import functools

import jax
import jax.numpy as jnp
from jax import lax
from jax.experimental import pallas as pl
from jax.experimental.pallas import tpu as pltpu
from jax.experimental.pallas import tpu_sc as plsc

N = 10000
E = 320000
D = 128

NC = 2
NS = 16
L = 16
NW = NC * NS
EPW = E // NW
CHA = 2000
NCHA = EPW // CHA
CHB = 128
NCHB = EPW // CHB
TAIL = EPW - NCHB * CHB
TAIL_OFF = EPW - TAIL
RPT = N // NS

_mesh = plsc.VectorSubcoreMesh(core_axis_name="c", subcore_axis_name="s")

_sc_params = pltpu.CompilerParams()
if "needs_layout_passes" in pltpu.CompilerParams.__dataclass_fields__:
    _sc_params = dataclasses.replace(_sc_params, needs_layout_passes=False)


def _sc_deg(src_hbm, w_hbm, out_hbm, deg_v, src_v0, src_v1, w_v0, w_v1,
            isem0, isem1):
    c = lax.axis_index("c")
    s = lax.axis_index("s")
    base0 = (c * NS + s) * EPW

    srcs = (src_v0, src_v1)
    ws = (w_v0, w_v1)
    isems = (isem0, isem1)

    for cix in (0, 1):
        pltpu.async_copy(src_hbm.at[pl.ds(base0 + cix * CHA, CHA)],
                         srcs[cix], isems[cix])
        pltpu.async_copy(w_hbm.at[pl.ds(base0 + cix * CHA, CHA)],
                         ws[cix], isems[cix])

    @pl.loop(0, N, step=L)
    def _(r):
        deg_v[pl.ds(r, L)] = jnp.zeros((L,), jnp.float32)

    for cix in range(NCHA):
        b = cix % 2
        pltpu.make_async_copy(
            src_hbm.at[pl.ds(base0, CHA)], srcs[b], isems[b]).wait()
        pltpu.make_async_copy(
            w_hbm.at[pl.ds(base0, CHA)], ws[b], isems[b]).wait()

        @pl.loop(0, CHA, step=L)
        def _(j):
            s16 = srcs[b][pl.ds(j, L)]
            w16 = ws[b][pl.ds(j, L)]
            plsc.addupdate_scatter(deg_v, [s16], w16)

        if cix + 2 < NCHA:
            nbase = base0 + (cix + 2) * CHA
            pltpu.async_copy(src_hbm.at[pl.ds(nbase, CHA)], srcs[b],
                             isems[b])
            pltpu.async_copy(w_hbm.at[pl.ds(nbase, CHA)], ws[b], isems[b])

    pltpu.sync_copy(deg_v, out_hbm.at[c, s])


def _sc_edge(src_hbm, dst_hbm, w_hbm, dis_hbm, h_hbm, z_hbm, out_hbm, acc_sh,
             src_v0, src_v1, dst_v0, dst_v1, w_v0, w_v1, wo_v0, wo_v1,
             dsc_v0, dsc_v1, rows_v0, rows_v1,
             dis_v, st_v, dt_v, wt_v, rt_v,
             isem0, isem1, gsem0, gsem1, osem0, osem1):
    c = lax.axis_index("c")
    s = lax.axis_index("s")
    base0 = (c * NS + s) * EPW

    pltpu.sync_copy(z_hbm, acc_sh.at[pl.ds(s * RPT, RPT)])
    pltpu.sync_copy(dis_hbm.at[0], dis_v)
    plsc.subcore_barrier()

    srcs = (src_v0, src_v1)
    dsts = (dst_v0, dst_v1)
    ws = (w_v0, w_v1)
    wos = (wo_v0, wo_v1)
    dscs = (dsc_v0, dsc_v1)
    rows = (rows_v0, rows_v1)
    isems = (isem0, isem1)
    gsems = (gsem0, gsem1)
    osems = (osem0, osem1)

    pltpu.sync_copy(src_hbm.at[pl.ds(base0, CHB)], src_v0)
    pltpu.sync_copy(dst_hbm.at[pl.ds(base0, CHB)], dst_v0)
    pltpu.sync_copy(w_hbm.at[pl.ds(base0, CHB)], w_v0)
    pltpu.async_copy(h_hbm.at[src_v0], rows_v0, gsem0)
    pltpu.async_copy(src_hbm.at[pl.ds(base0 + CHB, CHB)], src_v1, isem1)
    pltpu.async_copy(dst_hbm.at[pl.ds(base0 + CHB, CHB)], dst_v1, isem1)
    pltpu.async_copy(w_hbm.at[pl.ds(base0 + CHB, CHB)], w_v1, isem1)

    @pl.loop(0, NCHB // 2)
    def _(gg):
        for b in (0, 1):
            cix = 2 * gg + b
            nb = 1 - b

            @pl.when(cix >= 1)
            def _():
                pltpu.make_async_copy(
                    rows[nb], acc_sh.at[dscs[nb]], osems[nb]).wait()

            @pl.when(cix + 1 < NCHB)
            def _():
                pltpu.make_async_copy(
                    src_hbm.at[pl.ds(base0, CHB)], srcs[nb], isems[nb]).wait()
                pltpu.make_async_copy(
                    dst_hbm.at[pl.ds(base0, CHB)], dsts[nb], isems[nb]).wait()
                pltpu.make_async_copy(
                    w_hbm.at[pl.ds(base0, CHB)], ws[nb], isems[nb]).wait()
                pltpu.async_copy(h_hbm.at[srcs[nb]], rows[nb], gsems[nb])

            @pl.loop(0, CHB, step=L)
            def _(j):
                s16 = srcs[b][pl.ds(j, L)]
                d16 = dsts[b][pl.ds(j, L)]
                w16 = ws[b][pl.ds(j, L)]
                dis_s = plsc.load_gather(dis_v, [s16])
                dis_d = plsc.load_gather(dis_v, [d16])
                wos[b][pl.ds(j, L)] = -(dis_s * w16 * dis_d)
                dscs[b][pl.ds(j, L)] = d16

            pltpu.make_async_copy(
                h_hbm.at[srcs[b]], rows[b], gsems[b]).wait()

            @pl.loop(0, CHB, step=L)
            def _(j):
                wo16 = wos[b][pl.ds(j, L)]
                for i in range(L):
                    sc = wo16[i]
                    for k in range(D // L):
                        rows[b][j + i, pl.ds(k * L, L)] = (
                            rows[b][j + i, pl.ds(k * L, L)] * sc)

            pltpu.async_copy(rows[b], acc_sh.at[dscs[b]], osems[b],
                             add=True)

            @pl.when(cix + 2 < NCHB)
            def _():
                nbase = base0 + (cix + 2) * CHB
                pltpu.async_copy(src_hbm.at[pl.ds(nbase, CHB)], srcs[b],
                                 isems[b])
                pltpu.async_copy(dst_hbm.at[pl.ds(nbase, CHB)], dsts[b],
                                 isems[b])
                pltpu.async_copy(w_hbm.at[pl.ds(nbase, CHB)], ws[b], isems[b])

    pltpu.make_async_copy(rows_v1, acc_sh.at[dsc_v1], osem1).wait()

    pltpu.sync_copy(src_hbm.at[pl.ds(base0 + TAIL_OFF, TAIL)], st_v)
    pltpu.sync_copy(dst_hbm.at[pl.ds(base0 + TAIL_OFF, TAIL)], dt_v)
    pltpu.sync_copy(w_hbm.at[pl.ds(base0 + TAIL_OFF, TAIL)], wt_v)
    pltpu.sync_copy(h_hbm.at[st_v], rt_v)
    s16 = st_v[...]
    d16 = dt_v[...]
    w16 = wt_v[...]
    wo16 = -(plsc.load_gather(dis_v, [s16]) * w16
             * plsc.load_gather(dis_v, [d16]))
    for i in range(L):
        sc = wo16[i]
        for k in range(D // L):
            rt_v[i, pl.ds(k * L, L)] = rt_v[i, pl.ds(k * L, L)] * sc
    pltpu.sync_copy(rt_v, acc_sh.at[dt_v], add=True)

    plsc.subcore_barrier()
    pltpu.sync_copy(acc_sh.at[pl.ds(s * RPT, RPT)], out_hbm.at[c, s])


def _tc_dis(degw_ref, dis_ref):
    deg = jnp.sum(degw_ref[...], axis=0)
    dis_ref[...] = jnp.where(deg > 0, lax.rsqrt(deg), 0.0)[None, :]


BLK = 1000

_dot = functools.partial(jnp.dot, preferred_element_type=jnp.float32,
                         precision=lax.Precision.HIGHEST)


def _tc_prexh(x_ref, h_ref, wx_ref, w0_ref, b_ref, pre_ref):
    pre_ref[...] = (_dot(x_ref[...], wx_ref[...])
                    + _dot(h_ref[...], w0_ref[...]) + b_ref[...])


def _tc_dense(pxh_ref, c_ref, p_ref, w1_ref, wp_ref, wl_ref, bl_ref,
              ho_ref, H_ref, C_ref):
    cc = c_ref[...]
    tx1 = p_ref[0] + p_ref[1]
    pre = pxh_ref[...] + _dot(tx1, w1_ref[...])
    wp = wp_ref[...]
    gi = jax.nn.sigmoid(pre[:, 0:D] + wp[0:1] * cc)
    gf = jax.nn.sigmoid(pre[:, D:2 * D] + wp[1:2] * cc)
    gt = jnp.tanh(pre[:, 2 * D:3 * D])
    cn = gf * cc + gi * gt
    go = jax.nn.sigmoid(pre[:, 3 * D:4 * D] + wp[2:3] * cn)
    hn = go * jnp.tanh(cn)
    C_ref[...] = cn
    H_ref[...] = hn
    ho_ref[...] = (jnp.sum(jax.nn.relu(hn) * wl_ref[...], axis=1,
                           keepdims=True) + bl_ref[0, 0])


def kernel(x, edge_index, edge_weight, h, c, W_x, conv_W0, conv_W1, conv_b,
           w_peep, b_gate, W_lin, b_lin):
    src = edge_index[0]
    dst = edge_index[1]
    wm = jnp.where(src == dst, 0.0, edge_weight)

    deg_fn = pl.kernel(
        _sc_deg,
        out_type=jax.ShapeDtypeStruct((NC, NS, N), jnp.float32),
        mesh=_mesh,
        scratch_types=[
            pltpu.VMEM((N,), jnp.float32),
            pltpu.VMEM((CHA,), jnp.int32),
            pltpu.VMEM((CHA,), jnp.int32),
            pltpu.VMEM((CHA,), jnp.float32),
            pltpu.VMEM((CHA,), jnp.float32),
            pltpu.SemaphoreType.DMA,
            pltpu.SemaphoreType.DMA,
        ],
        compiler_params=_sc_params,
    )
    degw = deg_fn(src, wm).reshape(NC * NS, N)

    dis = pl.pallas_call(
        _tc_dis,
        out_shape=jax.ShapeDtypeStruct((1, N), jnp.float32),
    )(degw)

    zeros_blk = jnp.zeros((RPT, D), jnp.float32)

    edge_fn = pl.kernel(
        _sc_edge,
        out_type=jax.ShapeDtypeStruct((NC, NS, RPT, D), jnp.float32),
        mesh=_mesh,
        scratch_types=[
            pltpu.VMEM_SHARED((N, D), jnp.float32),
            pltpu.VMEM((CHB,), jnp.int32),
            pltpu.VMEM((CHB,), jnp.int32),
            pltpu.VMEM((CHB,), jnp.int32),
            pltpu.VMEM((CHB,), jnp.int32),
            pltpu.VMEM((CHB,), jnp.float32),
            pltpu.VMEM((CHB,), jnp.float32),
            pltpu.VMEM((CHB,), jnp.float32),
            pltpu.VMEM((CHB,), jnp.float32),
            pltpu.VMEM((CHB,), jnp.int32),
            pltpu.VMEM((CHB,), jnp.int32),
            pltpu.VMEM((CHB, D), jnp.float32),
            pltpu.VMEM((CHB, D), jnp.float32),
            pltpu.VMEM((N,), jnp.float32),
            pltpu.VMEM((TAIL,), jnp.int32),
            pltpu.VMEM((TAIL,), jnp.int32),
            pltpu.VMEM((TAIL,), jnp.float32),
            pltpu.VMEM((TAIL, D), jnp.float32),
            pltpu.SemaphoreType.DMA,
            pltpu.SemaphoreType.DMA,
            pltpu.SemaphoreType.DMA,
            pltpu.SemaphoreType.DMA,
            pltpu.SemaphoreType.DMA,
            pltpu.SemaphoreType.DMA,
        ],
        compiler_params=_sc_params,
    )
    parts = edge_fn(src, dst, wm, dis, h, zeros_blk).reshape(NC, N, D)

    wx_cat = jnp.transpose(W_x, (1, 0, 2)).reshape(D, 4 * D)
    w0_cat = jnp.transpose(conv_W0, (1, 0, 2)).reshape(D, 4 * D)
    w1_cat = jnp.transpose(conv_W1, (1, 0, 2)).reshape(D, 4 * D)
    b_cat = (conv_b + b_gate).reshape(1, 4 * D)
    wl_row = W_lin.reshape(1, D)
    bl = b_lin.reshape(1, 1)

    grid = (N // BLK,)
    pre_xh = pl.pallas_call(
        _tc_prexh,
        grid=grid,
        in_specs=[
            pl.BlockSpec((BLK, D), lambda i: (i, 0)),
            pl.BlockSpec((BLK, D), lambda i: (i, 0)),
            pl.BlockSpec((D, 4 * D), lambda i: (0, 0)),
            pl.BlockSpec((D, 4 * D), lambda i: (0, 0)),
            pl.BlockSpec((1, 4 * D), lambda i: (0, 0)),
        ],
        out_specs=pl.BlockSpec((BLK, 4 * D), lambda i: (i, 0)),
        out_shape=jax.ShapeDtypeStruct((N, 4 * D), jnp.float32),
    )(x, h, wx_cat, w0_cat, b_cat)

    h_out, H, C = pl.pallas_call(
        _tc_dense,
        grid=grid,
        in_specs=[
            pl.BlockSpec((BLK, 4 * D), lambda i: (i, 0)),
            pl.BlockSpec((BLK, D), lambda i: (i, 0)),
            pl.BlockSpec((NC, BLK, D), lambda i: (0, i, 0)),
            pl.BlockSpec((D, 4 * D), lambda i: (0, 0)),
            pl.BlockSpec((3, D), lambda i: (0, 0)),
            pl.BlockSpec((1, D), lambda i: (0, 0)),
            pl.BlockSpec((1, 1), lambda i: (0, 0)),
        ],
        out_specs=[
            pl.BlockSpec((BLK, 1), lambda i: (i, 0)),
            pl.BlockSpec((BLK, D), lambda i: (i, 0)),
            pl.BlockSpec((BLK, D), lambda i: (i, 0)),
        ],
        out_shape=[
            jax.ShapeDtypeStruct((N, 1), jnp.float32),
            jax.ShapeDtypeStruct((N, D), jnp.float32),
            jax.ShapeDtypeStruct((N, D), jnp.float32),
        ],
    )(pre_xh, c, parts, w1_cat, w_peep, wl_row, bl)

    return (h_out, H, C)
